# baseline (speedup 1.0000x reference)
import functools

import jax
import jax.numpy as jnp
from jax import lax
from jax.experimental import pallas as pl
from jax.experimental.pallas import tpu as pltpu
from jax.experimental.pallas import tpu_sc as plsc

_B, _D, _R = 64, 100000, 16
_NC, _NS, _L = 2, 16, 16
_NW = _NC * _NS
_ROWS_PER_W = _B // _NW
_C = 10000
_NCHUNK = _D // _C
_VPC = _C // _L
_NEG_INF = float("-inf")
_LN2 = 0.6931471805599453
_SQRT2 = 1.4142135


def _logv(a):
    bits = lax.bitcast_convert_type(a, jnp.int32)
    e = (bits >> 23) - 127
    m = lax.bitcast_convert_type(
        (bits & 0x007FFFFF) | 0x3F800000, jnp.float32)
    big = m > _SQRT2
    m = jnp.where(big, m * 0.5, m)
    e = (e + big.astype(jnp.int32)).astype(jnp.float32)
    s = (m - 1.0) / (m + 1.0)
    z = s * s
    p = 1.0 + z * (jnp.float32(1 / 3) + z * (jnp.float32(1 / 5)
        + z * (jnp.float32(1 / 7) + z * jnp.float32(1 / 9))))
    return e * jnp.float32(_LN2) + 2.0 * s * p


def _prep_body(t_ref, g_ref, s_ref):
    t = t_ref[...]
    et = jnp.exp(t)
    g_ref[...] = jnp.exp(-t) - et
    s_ref[0, 0] = jnp.sum(et)


def _prep(theta):
    g, s = pl.pallas_call(
        _prep_body,
        out_shape=(
            jax.ShapeDtypeStruct((1, _D), jnp.float32),
            jax.ShapeDtypeStruct((1, 1), jnp.float32),
        ),
        out_specs=(
            pl.BlockSpec(memory_space=pltpu.VMEM),
            pl.BlockSpec(memory_space=pltpu.SMEM),
        ),
    )(theta.reshape(1, _D))
    return g.reshape(_D), s.reshape(())


def _sc_body(x_hbm, g_hbm, th_hbm, gg_hbm, sp_hbm, u_hbm, out_hbm,
             xb0, xb1, gb0, gb1, tb0, tb1, Gb0, Gb1,
             uv, spv, idxv, valv,
             insem0, insem1, outsem0, outsem1, ssem):
    xbufs, gbufs, tbufs, Gbufs = (xb0, xb1), (gb0, gb1), (tb0, tb1), (Gb0, Gb1)
    insems, outsems = (insem0, insem1), (outsem0, outsem1)
    wid = lax.axis_index("s") * _NC + lax.axis_index("c")
    iota16 = lax.iota(jnp.int32, _L)

    pltpu.sync_copy(u_hbm, uv)
    pltpu.sync_copy(sp_hbm, spv)
    splus = spv[...][0]

    def start_in(b, c, row):
        off = row * _D + c * _C
        sem = insems[b]
        return (
            pltpu.async_copy(x_hbm.at[pl.ds(off, _C)], xbufs[b], sem),
            pltpu.async_copy(g_hbm.at[pl.ds(off, _C)], gbufs[b], sem),
            pltpu.async_copy(th_hbm.at[pl.ds(c * _C, _C)], tbufs[b], sem),
            pltpu.async_copy(gg_hbm.at[pl.ds(c * _C, _C)], Gbufs[b], sem),
        )

    def start_out(b, c, row):
        off = row * _D + c * _C
        return pltpu.async_copy(
            xbufs[b], out_hbm.at[pl.ds(off, _C)], outsems[b])

    for r in range(_ROWS_PER_W):
        row = wid * _ROWS_PER_W + r
        row_base = (row * _D) << 1

        carry = (
            jnp.full((_L,), _NEG_INF, jnp.float32),
            jnp.zeros((_L,), jnp.int32),
            jnp.zeros((_L,), jnp.float32),
            jnp.float32(_NEG_INF),
            jnp.zeros((_L,), jnp.float32),
        )

        pend_in = {0: start_in(0, 0, row)}
        pend_out = {}
        for c in range(_NCHUNK):
            b = c & 1
            if c + 1 < _NCHUNK:
                nb = 1 - b
                if nb in pend_out:
                    pend_out.pop(nb).wait()
                pend_in[nb] = start_in(nb, c + 1, row)
            for d in pend_in.pop(b):
                d.wait()

            xrow, grow, trow, Grow = xbufs[b], gbufs[b], tbufs[b], Gbufs[b]
            cbase = row_base + ((c * _C) << 1)

            def chunk_body(v, cr, xrow=xrow, grow=grow, trow=trow,
                           Grow=Grow, cbase=cbase):
                K, I, F, thr, accG = cr
                sl = pl.ds(v * _L, _L)
                xv = xrow[sl]
                gv = grow[sl]
                tv = trow[sl]
                Gv = Grow[sl]
                t1 = xv * tv
                fd = tv - (t1 + t1)
                key = fd + gv
                accG = accG + xv * Gv
                m = key > thr

                def merge(ops):
                    K, I, F, key, fd, m, xv, v = ops
                    enc = (cbase + ((v * _L + iota16) << 1)) | xv.astype(jnp.int32)
                    km = jnp.where(m, key, _NEG_INF)
                    ck, ci = plsc.sort_key_val(km, enc, descending=False)
                    _, cf = plsc.sort_key_val(km, fd, descending=False)
                    sel = K >= ck
                    nK = jnp.where(sel, K, ck)
                    nI = jnp.where(sel, I, ci)
                    nF = jnp.where(sel, F, cf)
                    sK, sI = plsc.sort_key_val(nK, nI, descending=True)
                    _, sF = plsc.sort_key_val(nK, nF, descending=True)
                    return sK, sI, sF, jnp.min(sK)

                def keep(ops):
                    K, I, F, key, fd, m, xv, v = ops
                    return K, I, F, thr

                K, I, F, thr = lax.cond(
                    jnp.any(m), merge, keep, (K, I, F, key, fd, m, xv, v))
                return K, I, F, thr, accG

            carry = lax.fori_loop(0, _VPC, chunk_body, carry)
            pend_out[b] = start_out(b, c, row)

        for b in sorted(pend_out):
            pend_out.pop(b).wait()

        K, I, F, _, accG = carry

        sx = splus + jnp.sum(accG)
        corr = jnp.sum(jnp.exp(-F) - jnp.exp(F))
        sy = sx + corr
        lse_x = _logv(jnp.full((_L,), sx, jnp.float32))
        lse_y = _logv(jnp.full((_L,), sy, jnp.float32))

        Kw = K
        ordF = jnp.zeros((_L,), jnp.float32)
        for j in range(_R):
            mx = jnp.max(Kw)
            imin = jnp.min(jnp.where(Kw == mx, I, jnp.int32(2**31 - 1)))
            pick = I == imin
            fdj = jnp.sum(jnp.where(pick, F, 0.0))
            ordF = jnp.where(iota16 == j, fdj, ordF)
            Kw = jnp.where(pick, jnp.float32(_NEG_INF), Kw)

        def wo_repl_logp(ls):
            mxv = jnp.max(ls)
            cum = plsc.cumsum(jnp.exp(ls - mxv))
            lu = mxv + _logv(cum)
            w = jnp.exp(lu)
            return jnp.sum(ls - _logv(1.0 - w))

        log_x = wo_repl_logp(ordF - lse_x)
        log_y = wo_repl_logp((-ordF) - lse_y)
        log_acc = jnp.sum(F) + log_y - log_x
        u_vec = uv[pl.ds((row >> 4) << 4, _L)]
        lane = row & (_L - 1)
        accept = jnp.any(
            (iota16 == lane)
            & (jnp.exp(jnp.full((_L,), log_acc)) >= u_vec))

        @pl.when(accept)
        def _scatter():
            idxv[...] = I >> 1
            valv[...] = 1.0 - (I & 1).astype(jnp.float32)
            pltpu.async_copy(valv, out_hbm.at[idxv], ssem).wait()


@functools.cache
def _get_sc_sampler():
    mesh = plsc.VectorSubcoreMesh(
        core_axis_name="c", subcore_axis_name="s",
        num_cores=_NC, num_subcores=_NS)
    return pl.kernel(
        _sc_body,
        out_type=jax.ShapeDtypeStruct((_B * _D,), jnp.float32),
        mesh=mesh,
        scratch_types=(
            [pltpu.VMEM((_C,), jnp.float32)] * 8
            + [
                pltpu.VMEM((_B,), jnp.float32),
                pltpu.VMEM((_L,), jnp.float32),
                pltpu.VMEM((_L,), jnp.int32),
                pltpu.VMEM((_L,), jnp.float32),
            ]
            + [pltpu.SemaphoreType.DMA] * 5
        ),
        compiler_params=pltpu.CompilerParams(needs_layout_passes=False),
    )


def kernel(x, theta, gumbel, u):
    gg, splus = _prep(theta)
    sp = jnp.broadcast_to(splus.reshape(1), (_L,))
    out = _get_sc_sampler()(
        x.reshape(_B * _D), gumbel.reshape(_B * _D), theta, gg, sp, u)
    return out.reshape(_B, _D)


# speedup vs baseline: 2.4315x; 2.4315x over previous
import functools

import jax
import jax.numpy as jnp
from jax import lax
from jax.experimental import pallas as pl
from jax.experimental.pallas import tpu as pltpu
from jax.experimental.pallas import tpu_sc as plsc

_B, _D, _R = 64, 100000, 16
_NC, _NS, _L = 2, 16, 16
_NW = _NC * _NS
_ROWS_PER_W = _B // _NW
_C = 10000
_NCHUNK = _D // _C
_VPC = _C // _L
_BLK = 25
_BPC = _VPC // _BLK
_NEG_INF = float("-inf")
_LN2 = 0.6931471805599453
_SQRT2 = 1.4142135


def _logv(a):
    bits = lax.bitcast_convert_type(a, jnp.int32)
    e = (bits >> 23) - 127
    m = lax.bitcast_convert_type(
        (bits & 0x007FFFFF) | 0x3F800000, jnp.float32)
    big = m > _SQRT2
    m = jnp.where(big, m * 0.5, m)
    e = (e + big.astype(jnp.int32)).astype(jnp.float32)
    s = (m - 1.0) / (m + 1.0)
    z = s * s
    p = 1.0 + z * (jnp.float32(1 / 3) + z * (jnp.float32(1 / 5)
        + z * (jnp.float32(1 / 7) + z * jnp.float32(1 / 9))))
    return e * jnp.float32(_LN2) + 2.0 * s * p


def _sc_body(x_hbm, g_hbm, th_hbm, u_hbm, out_hbm,
             xb0, xb1, gb0, gb1, tb0, tb1,
             uv, idxv, valv,
             insem0, insem1, outsem0, outsem1, ssem):
    xbufs, gbufs, tbufs = (xb0, xb1), (gb0, gb1), (tb0, tb1)
    insems, outsems = (insem0, insem1), (outsem0, outsem1)
    wid = lax.axis_index("s") * _NC + lax.axis_index("c")
    iota16 = lax.iota(jnp.int32, _L)

    pltpu.sync_copy(u_hbm, uv)

    def start_in(b, c, row):
        off = row * _D + c * _C
        sem = insems[b]
        return (
            pltpu.async_copy(x_hbm.at[pl.ds(off, _C)], xbufs[b], sem),
            pltpu.async_copy(g_hbm.at[pl.ds(off, _C)], gbufs[b], sem),
            pltpu.async_copy(th_hbm.at[pl.ds(c * _C, _C)], tbufs[b], sem),
        )

    def start_out(b, c, row):
        off = row * _D + c * _C
        return pltpu.async_copy(
            xbufs[b], out_hbm.at[pl.ds(off, _C)], outsems[b])

    def tree(vals, op):
        while len(vals) > 1:
            nxt = [op(vals[j], vals[j + 1]) for j in range(0, len(vals) - 1, 2)]
            if len(vals) % 2:
                nxt.append(vals[-1])
            vals = nxt
        return vals[0]

    def row_body(r, _carry_unused):
        row = wid * _ROWS_PER_W + r
        row_base = (row * _D) << 1

        K = jnp.full((_L,), _NEG_INF, jnp.float32)
        I = jnp.zeros((_L,), jnp.int32)
        F = jnp.zeros((_L,), jnp.float32)
        thr = jnp.float32(_NEG_INF)
        accE = jnp.zeros((_L,), jnp.float32)

        pend_in = {0: start_in(0, 0, row)}
        pend_out = {}
        for c in range(_NCHUNK):
            b = c & 1
            if c + 1 < _NCHUNK:
                nb = 1 - b
                if nb in pend_out:
                    pend_out.pop(nb).wait()
                pend_in[nb] = start_in(nb, c + 1, row)
            for d in pend_in.pop(b):
                d.wait()
            pend_out[b] = start_out(b, c, row)

            xrow, grow, trow = xbufs[b], gbufs[b], tbufs[b]
            cbase = row_base + ((c * _C) << 1)

            def block_body(bk, cr, xrow=xrow, grow=grow, trow=trow,
                           cbase=cbase):
                K, I, F, thr, accE = cr
                keys, exps = [], []
                for i in range(_BLK):
                    sl = pl.ds((bk * _BLK + i) * _L, _L)
                    xv = xrow[sl]
                    tv = trow[sl]
                    gv = grow[sl]
                    t1 = xv * tv
                    fd = tv - (t1 + t1)
                    keys.append(fd + gv)
                    exps.append(jnp.exp(fd))
                accE = accE + tree(exps, lambda a, b2: a + b2)
                bm = tree(keys, jnp.maximum)

                def rescan(ops):
                    K, I, F, thr = ops

                    def vbody(v, cr2):
                        K, I, F, thr = cr2
                        sl = pl.ds(v * _L, _L)
                        xv = xrow[sl]
                        tv = trow[sl]
                        gv = grow[sl]
                        t1 = xv * tv
                        fd = tv - (t1 + t1)
                        key = fd + gv
                        m = key > thr

                        def merge(ops2):
                            K, I, F, key, fd, m, xv, v = ops2
                            enc = (cbase + ((v * _L + iota16) << 1)) \
                                | xv.astype(jnp.int32)
                            km = jnp.where(m, key, _NEG_INF)
                            ck, ci = plsc.sort_key_val(km, enc,
                                                       descending=False)
                            _, cf = plsc.sort_key_val(km, fd,
                                                      descending=False)
                            sel = K >= ck
                            nK = jnp.where(sel, K, ck)
                            nI = jnp.where(sel, I, ci)
                            nF = jnp.where(sel, F, cf)
                            sK, sI = plsc.sort_key_val(nK, nI,
                                                       descending=True)
                            _, sF = plsc.sort_key_val(nK, nF,
                                                      descending=True)
                            return sK, sI, sF, jnp.min(sK)

                        def keep(ops2):
                            K, I, F, key, fd, m, xv, v = ops2
                            return K, I, F, thr

                        K, I, F, thr = lax.cond(
                            jnp.any(m), merge, keep,
                            (K, I, F, key, fd, m, xv, v))
                        return K, I, F, thr

                    return lax.fori_loop(
                        bk * _BLK, bk * _BLK + _BLK, vbody, (K, I, F, thr))

                def norescan(ops):
                    return ops

                K, I, F, thr = lax.cond(
                    jnp.any(bm > thr), rescan, norescan, (K, I, F, thr))
                return K, I, F, thr, accE

            K, I, F, thr, accE = lax.fori_loop(
                0, _BPC, block_body, (K, I, F, thr, accE))

        for b in sorted(pend_out):
            pend_out.pop(b).wait()

        sx = jnp.sum(accE)
        corr = jnp.sum(jnp.exp(-F) - jnp.exp(F))
        sy = sx + corr
        lse_x = _logv(jnp.full((_L,), sx, jnp.float32))
        lse_y = _logv(jnp.full((_L,), sy, jnp.float32))

        Kw = K
        ordF = jnp.zeros((_L,), jnp.float32)
        for j in range(_R):
            mx = jnp.max(Kw)
            imin = jnp.min(jnp.where(Kw == mx, I, jnp.int32(2**31 - 1)))
            pick = I == imin
            fdj = jnp.sum(jnp.where(pick, F, 0.0))
            ordF = jnp.where(iota16 == j, fdj, ordF)
            Kw = jnp.where(pick, jnp.float32(_NEG_INF), Kw)

        def wo_repl_logp(ls):
            mxv = jnp.max(ls)
            cum = plsc.cumsum(jnp.exp(ls - mxv))
            lu = mxv + _logv(cum)
            w = jnp.exp(lu)
            return jnp.sum(ls - _logv(1.0 - w))

        log_x = wo_repl_logp(ordF - lse_x)
        log_y = wo_repl_logp((-ordF) - lse_y)
        log_acc = jnp.sum(F) + log_y - log_x
        u_vec = uv[pl.ds((row >> 4) << 4, _L)]
        lane = row & (_L - 1)
        accept = jnp.any(
            (iota16 == lane)
            & (jnp.exp(jnp.full((_L,), log_acc)) >= u_vec))

        @pl.when(accept)
        def _scatter():
            idxv[...] = I >> 1
            valv[...] = 1.0 - (I & 1).astype(jnp.float32)
            pltpu.async_copy(valv, out_hbm.at[idxv], ssem).wait()

        return 0

    lax.fori_loop(0, _ROWS_PER_W, row_body, 0)


@functools.cache
def _get_sc_sampler():
    mesh = plsc.VectorSubcoreMesh(
        core_axis_name="c", subcore_axis_name="s",
        num_cores=_NC, num_subcores=_NS)
    return pl.kernel(
        _sc_body,
        out_type=jax.ShapeDtypeStruct((_B * _D,), jnp.float32),
        mesh=mesh,
        scratch_types=(
            [pltpu.VMEM((_C,), jnp.float32)] * 6
            + [
                pltpu.VMEM((_B,), jnp.float32),
                pltpu.VMEM((_L,), jnp.int32),
                pltpu.VMEM((_L,), jnp.float32),
            ]
            + [pltpu.SemaphoreType.DMA] * 5
        ),
        compiler_params=pltpu.CompilerParams(needs_layout_passes=False),
    )


def kernel(x, theta, gumbel, u):
    out = _get_sc_sampler()(
        x.reshape(_B * _D), gumbel.reshape(_B * _D), theta, u)
    return out.reshape(_B, _D)
